# baseline (speedup 1.0000x reference)
import dataclasses
import functools

import jax
import jax.numpy as jnp
from jax import lax
from jax.experimental import pallas as pl
from jax.experimental.pallas import tpu as pltpu
from jax.experimental.pallas import tpu_sc as plsc

N = 10000
NP = 10240
D = 128
E_REAL = 320000
E_ALL = E_REAL + N
EPAD = 344064
PAD_EDGES = EPAD - E_ALL
ROWS_PER_TILE = NP // 16
EW = 128
SW = 64
GW = 16
NC = 64
CHUNKS = ROWS_PER_TILE // NC
EDGES_PER_TILE = EPAD // 16
WINDOWS = EDGES_PER_TILE // SW
GROUPS = WINDOWS // GW
N_ITER = 10
BLK = 256


def _tc_pre_body(h_ref, emb_ref, w1_ref, b1_ref, w2_ref, b2_ref, w3_ref,
                 b3_ref, o_ref):
    hv = h_ref[0, 0, :]
    ids = lax.broadcasted_iota(jnp.int32, (BLK, D), 1)
    oh = (hv[:, None] == ids).astype(jnp.float32)
    x = jnp.dot(oh, emb_ref[...], preferred_element_type=jnp.float32)
    x = jnp.maximum(
        jnp.dot(x, w1_ref[...], preferred_element_type=jnp.float32)
        + b1_ref[...][None, :], 0.0)
    x = jnp.maximum(
        jnp.dot(x, w2_ref[...], preferred_element_type=jnp.float32)
        + b2_ref[...][None, :], 0.0)
    x = (jnp.dot(x, w3_ref[...], preferred_element_type=jnp.float32)
         + b3_ref[...][None, :])
    o_ref[...] = x


def _tc_pre(h_pad, emb, W1, b1, W2, b2, W3, b3):
    grid = (NP // BLK,)
    full_w = pl.BlockSpec((D, D), lambda i: (0, 0))
    full_b = pl.BlockSpec((D,), lambda i: (0,))
    return pl.pallas_call(
        _tc_pre_body,
        grid=grid,
        in_specs=[
            pl.BlockSpec((1, 1, BLK), lambda i: (i, 0, 0)),
            full_w, full_w, full_b, full_w, full_b, full_w, full_b,
        ],
        out_specs=pl.BlockSpec((BLK, D), lambda i: (i, 0)),
        out_shape=jax.ShapeDtypeStruct((NP, D), jnp.float32),
    )(h_pad, emb, W1, b1, W2, b2, W3, b3)


def _tc_readout_body(x_ref, r0w_ref, r0b_ref, r1w_ref, r1b_ref, r2w_ref,
                     r2b_ref, o_ref):
    y = jnp.maximum(
        jnp.dot(x_ref[...], r0w_ref[...], preferred_element_type=jnp.float32)
        + r0b_ref[...][None, :], 0.0)
    y = jnp.maximum(
        jnp.dot(y, r1w_ref[...], preferred_element_type=jnp.float32)
        + r1b_ref[...][None, :], 0.0)
    y = (jnp.dot(y, r2w_ref[...], preferred_element_type=jnp.float32)
         + r2b_ref[...][None, :])
    o_ref[...] = y


def _tc_readout(x, R0w, R0b, R1w, R1b, R2w, R2b):
    grid = (NP // BLK,)
    return pl.pallas_call(
        _tc_readout_body,
        grid=grid,
        in_specs=[
            pl.BlockSpec((BLK, D), lambda i: (i, 0)),
            pl.BlockSpec((D, 64), lambda i: (0, 0)),
            pl.BlockSpec((64,), lambda i: (0,)),
            pl.BlockSpec((64, 32), lambda i: (0, 0)),
            pl.BlockSpec((32,), lambda i: (0,)),
            pl.BlockSpec((32, 16), lambda i: (0, 0)),
            pl.BlockSpec((16,), lambda i: (0,)),
        ],
        out_specs=pl.BlockSpec((BLK, 16), lambda i: (i, 0)),
        out_shape=jax.ShapeDtypeStruct((NP, 16), jnp.float32),
    )(x, R0w, R0b, R1w, R1b, R2w, R2b)


def _sc_body(src_hbm, dst2d_hbm, lp_hbm, hw_hbm, hb_hbm,
             x_hbm, steps_hbm, rem_hbm, pa_hbm,
             sbufB, dbufB, drow64, dlow64, rows0, rows1,
             degb5, dinvb, dscaleb, sumhb, contb, stepsb, rembuf,
             dotb, pbuf, ocontb, hwbuf, hbbuf,
             spdeg, agg_sh, gsem0, gsem1, ssem0, ssem1):
    cid = lax.axis_index("c")
    sid = lax.axis_index("s")

    @pl.when(cid == 0)
    def _work():
        nbase = sid * ROWS_PER_TILE
        ebase = sid * EDGES_PER_TILE
        ngroups = ROWS_PER_TILE // 16
        zero16 = jnp.zeros((16,), jnp.float32)
        one16 = jnp.full((16,), 1.0, jnp.float32)

        def _zero(buf):
            @pl.loop(0, EW)
            def _zr(i):
                for k in range(8):
                    buf[i, pl.ds(k * 16, 16)] = zero16

        _zero(rows0)
        _zero(rows1)

        @pl.loop(0, ngroups)
        def _st(i):
            sl = pl.ds(i * 16, 16)
            sumhb[sl] = zero16
            contb[sl] = one16
            stepsb[sl] = one16

        pltpu.sync_copy(hw_hbm, hwbuf)
        pltpu.sync_copy(hb_hbm, hbbuf)

        pltpu.sync_copy(rows0.at[pl.ds(0, 5)], spdeg.at[pl.ds(sid * 5, 5)])
        plsc.subcore_barrier()

        rows1h = rows1.at[pl.ds(0, SW)]

        @pl.loop(0, GROUPS)
        def _degg(g):
            pltpu.sync_copy(
                dst2d_hbm.at[pl.ds(sid * WINDOWS + g * GW, GW)], dbufB)
            for j in range(GW):
                for k in range(SW // 16):
                    sl = pl.ds(k * 16, 16)
                    v = dbufB[j, sl]
                    drow64[sl] = lax.shift_right_logical(v, 7)
                    dlow64[sl] = lax.bitwise_and(v, 127)

                @pl.loop(0, SW)
                def _set(i):
                    ii = jnp.zeros((16,), jnp.int32) + i
                    lw = plsc.load_gather(dlow64, [ii])
                    plsc.store_scatter(rows1, [ii, lw], one16)

                pltpu.sync_copy(rows1h, spdeg.at[drow64], add=True)

                @pl.loop(0, SW)
                def _clr(i):
                    ii = jnp.zeros((16,), jnp.int32) + i
                    lw = plsc.load_gather(dlow64, [ii])
                    plsc.store_scatter(rows1, [ii, lw], zero16)

        plsc.subcore_barrier()

        pltpu.sync_copy(spdeg.at[pl.ds(sid * 5, 5)], degb5.at[pl.ds(0, 5)])

        @pl.loop(0, 5)
        def _dinv(q):
            for k in range(8):
                deg = jnp.maximum(degb5[q, pl.ds(k * 16, 16)], 1.0)
                ii = lax.bitcast_convert_type(deg, jnp.int32)
                ii = jnp.int32(0x5F3759DF) - lax.shift_right_arithmetic(ii, 1)
                y = lax.bitcast_convert_type(ii, jnp.float32)
                half = deg * 0.5
                for _ in range(4):
                    y = y * (1.5 - half * y * y)
                sl = pl.ds(q * 128 + k * 16, 16)
                dinvb[sl] = y
                dscaleb[sl] = deg * y

        for c in range(CHUNKS):
            rb = nbase + c * NC
            sl_rows = pl.ds(rb, NC)
            pltpu.sync_copy(lp_hbm.at[sl_rows], rows0.at[pl.ds(0, NC)])

            @pl.loop(0, NC)
            def _scale(i, _c=c):
                bidx = jnp.zeros((16,), jnp.int32) + (_c * NC + i)
                dv = plsc.load_gather(dinvb, [bidx])
                for k in range(8):
                    sl = pl.ds(k * 16, 16)
                    rows0[i, sl] = rows0[i, sl] * dv

            pltpu.sync_copy(rows0.at[pl.ds(0, NC)], pa_hbm.at[sl_rows])
            pltpu.sync_copy(rows1.at[pl.ds(0, NC)], x_hbm.at[sl_rows])

        plsc.subcore_barrier()

        hw = [hwbuf[pl.ds(k * 16, 16)] for k in range(8)]
        hb16 = hbbuf[...]
        rows0h = rows0.at[pl.ds(0, SW)]

        @pl.loop(0, N_ITER)
        def _iter(t):
            _zero(rows0)
            for c in range(5):
                pltpu.sync_copy(rows0, agg_sh.at[pl.ds(nbase + c * EW, EW)])
            plsc.subcore_barrier()

            @pl.loop(0, GROUPS)
            def _grp(g):
                gb = ebase + g * (GW * SW)
                pltpu.sync_copy(src_hbm.at[pl.ds(gb, GW * SW)], sbufB)
                pltpu.sync_copy(
                    dst2d_hbm.at[pl.ds(sid * WINDOWS + g * GW, GW)], dbufB)
                for k in range(GW):
                    pltpu.async_copy(
                        pa_hbm.at[sbufB.at[pl.ds(k * SW, SW)]],
                        rows0h, gsem0).wait()
                    pltpu.sync_copy(rows0h, agg_sh.at[dbufB.at[k]], add=True)

            plsc.subcore_barrier()

            for c in range(CHUNKS):
                rb = nbase + c * NC
                sl_rows = pl.ds(rb, NC)
                la = pltpu.make_async_copy(
                    agg_sh.at[sl_rows], rows0.at[pl.ds(0, NC)], gsem0)
                lb = pltpu.make_async_copy(
                    pa_hbm.at[sl_rows], rows0.at[pl.ds(NC, NC)], gsem1)
                lc = pltpu.make_async_copy(
                    x_hbm.at[sl_rows], rows1.at[pl.ds(0, NC)], ssem0)
                la.start(); lb.start(); lc.start()
                la.wait(); lb.wait(); lc.wait()

                @pl.loop(0, NC)
                def _p1(i, _c=c):
                    bidx = jnp.zeros((16,), jnp.int32) + (_c * NC + i)
                    dv = plsc.load_gather(dinvb, [bidx])
                    acc = zero16
                    for k in range(8):
                        sl = pl.ds(k * 16, 16)
                        a = rows0[i, sl] * dv
                        rows0[i, sl] = a
                        acc = acc + a * hw[k]
                    plsc.store_scatter(
                        dotb, [jnp.zeros((16,), jnp.int32) + i],
                        one16 * jnp.sum(acc))

                @pl.loop(0, NC // 16)
                def _hc(gidx, _c=c):
                    lsl = pl.ds(gidx * 16, 16)
                    gsl = pl.ds(_c * NC + gidx * 16, 16)
                    d16 = dotb[lsl]
                    z = d16 + hb16
                    hh = 1.0 / (1.0 + jnp.exp(-z))
                    sh = sumhb[gsl]
                    ct = contb[gsl]
                    st = stepsb[gsl]
                    pm = jnp.where(sh + hh < 0.99, 1.0, 0.0) * ct
                    st = st + pm
                    sh = sh + pm * hh
                    fi = jnp.where(st < float(N_ITER), 1.0, 0.0)
                    cond = pm * fi
                    p = jnp.where(cond > 0.0, sh, 1.0 - sh)
                    stepsb[gsl] = st
                    sumhb[gsl] = sh
                    contb[gsl] = ct * pm
                    pbuf[lsl] = p
                    ocontb[lsl] = ct

                @pl.loop(0, NC)
                def _p2(i, _c=c):
                    gidx = jnp.zeros((16,), jnp.int32) + (_c * NC + i)
                    lidx = jnp.zeros((16,), jnp.int32) + i
                    dv = plsc.load_gather(dinvb, [gidx])
                    dsc = plsc.load_gather(dscaleb, [gidx])
                    p_s = plsc.load_gather(pbuf, [lidx])
                    sh_s = plsc.load_gather(sumhb, [gidx])
                    ct_s = plsc.load_gather(ocontb, [lidx])
                    for k in range(8):
                        sl = pl.ds(k * 16, 16)
                        new = rows0[i, sl]
                        old = rows0[NC + i, sl] * dsc
                        rows1[i, sl] = rows1[i, sl] + (new * p_s - old * sh_s) * ct_s
                        rows0[NC + i, sl] = new * dv

                pltpu.sync_copy(rows0.at[pl.ds(NC, NC)], pa_hbm.at[sl_rows])
                pltpu.sync_copy(rows1.at[pl.ds(0, NC)], x_hbm.at[sl_rows])

            plsc.subcore_barrier()

        @pl.loop(0, ngroups)
        def _rem(i):
            sl = pl.ds(i * 16, 16)
            rembuf[sl] = 1.0 - sumhb[sl]

        pltpu.sync_copy(stepsb, steps_hbm.at[pl.ds(nbase, ROWS_PER_TILE)])
        pltpu.sync_copy(rembuf, rem_hbm.at[pl.ds(nbase, ROWS_PER_TILE)])


def _sc_prop(src_all, dst_all, local_preds, hw, hb16):
    mesh = plsc.VectorSubcoreMesh(core_axis_name="c", subcore_axis_name="s")
    f32 = jnp.float32
    cp = pltpu.CompilerParams()
    if "needs_layout_passes" in pltpu.CompilerParams.__dataclass_fields__:
        cp = dataclasses.replace(cp, needs_layout_passes=False)
    kern = pl.kernel(
        _sc_body,
        compiler_params=cp,
        out_type=[
            jax.ShapeDtypeStruct((NP, D), f32),
            jax.ShapeDtypeStruct((NP,), f32),
            jax.ShapeDtypeStruct((NP,), f32),
            jax.ShapeDtypeStruct((NP, D), f32),
        ],
        mesh=mesh,
        scratch_types=[
            pltpu.VMEM((GW * SW,), jnp.int32),
            pltpu.VMEM((GW, SW), jnp.int32),
            pltpu.VMEM((SW,), jnp.int32),
            pltpu.VMEM((SW,), jnp.int32),
            pltpu.VMEM((EW, D), f32),
            pltpu.VMEM((EW, D), f32),
            pltpu.VMEM((8, D), f32),
            pltpu.VMEM((ROWS_PER_TILE,), f32),
            pltpu.VMEM((ROWS_PER_TILE,), f32),
            pltpu.VMEM((ROWS_PER_TILE,), f32),
            pltpu.VMEM((ROWS_PER_TILE,), f32),
            pltpu.VMEM((ROWS_PER_TILE,), f32),
            pltpu.VMEM((ROWS_PER_TILE,), f32),
            pltpu.VMEM((NC,), f32),
            pltpu.VMEM((NC,), f32),
            pltpu.VMEM((NC,), f32),
            pltpu.VMEM((D,), f32),
            pltpu.VMEM((16,), f32),
            pltpu.VMEM_SHARED((NP // 128, D), f32),
            pltpu.VMEM_SHARED((NP, D), f32),
            pltpu.SemaphoreType.DMA,
            pltpu.SemaphoreType.DMA,
            pltpu.SemaphoreType.DMA,
            pltpu.SemaphoreType.DMA,
        ],
    )
    dst2d = dst_all.reshape(EPAD // SW, SW)
    return kern(src_all, dst2d, local_preds, hw, hb16)


def kernel(g, h, e, snorm_n, snorm_e, emb, W1, b1, W2, b2, W3, b3,
           halt_w, halt_b, R0w, R0b, R1w, R1b, R2w, R2b):
    loop_idx = jnp.arange(N, dtype=jnp.int32)
    pad_idx = jnp.full((PAD_EDGES,), NP - 1, jnp.int32)
    src_all = jnp.concatenate([g[0], loop_idx, pad_idx])
    dst_all = jnp.concatenate([g[1], loop_idx, pad_idx])
    h_pad = jnp.concatenate([h, jnp.zeros((NP - N,), jnp.int32)])
    h_pad = h_pad.reshape(NP // BLK, 1, BLK)
    hw = halt_w.reshape(D)
    hb16 = jnp.full((16,), halt_b[0], jnp.float32)

    local_preds = _tc_pre(h_pad, emb, W1, b1, W2, b2, W3, b3)
    x, steps, rem, _ = _sc_prop(src_all, dst_all, local_preds, hw, hb16)
    y = _tc_readout(x, R0w, R0b, R1w, R1b, R2w, R2b)
    return y[:N], steps[:N], rem[:N]


# speedup vs baseline: 1.4261x; 1.4261x over previous
import dataclasses
import functools

import jax
import jax.numpy as jnp
from jax import lax
from jax.experimental import pallas as pl
from jax.experimental.pallas import tpu as pltpu
from jax.experimental.pallas import tpu_sc as plsc

N = 10000
NP = 10240
D = 128
E_REAL = 320000
E_ALL = E_REAL + N
EPAD = 331776
PAD_EDGES = EPAD - E_ALL
ROWS_PER_TILE = NP // 16
EW = 64
NC = 64
CHUNKS = ROWS_PER_TILE // NC
EDGES_PER_TILE = EPAD // 16
WINDOWS = EDGES_PER_TILE // EW
N_ITER = 10
BLK = 256


def _tc_pre_body(h_ref, emb_ref, w1_ref, b1_ref, w2_ref, b2_ref, w3_ref,
                 b3_ref, o_ref):
    hv = h_ref[0, 0, :]
    ids = lax.broadcasted_iota(jnp.int32, (BLK, D), 1)
    oh = (hv[:, None] == ids).astype(jnp.float32)
    x = jnp.dot(oh, emb_ref[...], preferred_element_type=jnp.float32)
    x = jnp.maximum(
        jnp.dot(x, w1_ref[...], preferred_element_type=jnp.float32)
        + b1_ref[...][None, :], 0.0)
    x = jnp.maximum(
        jnp.dot(x, w2_ref[...], preferred_element_type=jnp.float32)
        + b2_ref[...][None, :], 0.0)
    x = (jnp.dot(x, w3_ref[...], preferred_element_type=jnp.float32)
         + b3_ref[...][None, :])
    o_ref[...] = x


def _tc_pre(h_pad, emb, W1, b1, W2, b2, W3, b3):
    grid = (NP // BLK,)
    full_w = pl.BlockSpec((D, D), lambda i: (0, 0))
    full_b = pl.BlockSpec((D,), lambda i: (0,))
    return pl.pallas_call(
        _tc_pre_body,
        grid=grid,
        in_specs=[
            pl.BlockSpec((1, 1, BLK), lambda i: (i, 0, 0)),
            full_w, full_w, full_b, full_w, full_b, full_w, full_b,
        ],
        out_specs=pl.BlockSpec((BLK, D), lambda i: (i, 0)),
        out_shape=jax.ShapeDtypeStruct((NP, D), jnp.float32),
    )(h_pad, emb, W1, b1, W2, b2, W3, b3)


def _tc_readout_body(x_ref, r0w_ref, r0b_ref, r1w_ref, r1b_ref, r2w_ref,
                     r2b_ref, o_ref):
    y = jnp.maximum(
        jnp.dot(x_ref[...], r0w_ref[...], preferred_element_type=jnp.float32)
        + r0b_ref[...][None, :], 0.0)
    y = jnp.maximum(
        jnp.dot(y, r1w_ref[...], preferred_element_type=jnp.float32)
        + r1b_ref[...][None, :], 0.0)
    y = (jnp.dot(y, r2w_ref[...], preferred_element_type=jnp.float32)
         + r2b_ref[...][None, :])
    o_ref[...] = y


def _tc_readout(x, R0w, R0b, R1w, R1b, R2w, R2b):
    grid = (NP // BLK,)
    return pl.pallas_call(
        _tc_readout_body,
        grid=grid,
        in_specs=[
            pl.BlockSpec((BLK, D), lambda i: (i, 0)),
            pl.BlockSpec((D, 64), lambda i: (0, 0)),
            pl.BlockSpec((64,), lambda i: (0,)),
            pl.BlockSpec((64, 32), lambda i: (0, 0)),
            pl.BlockSpec((32,), lambda i: (0,)),
            pl.BlockSpec((32, 16), lambda i: (0, 0)),
            pl.BlockSpec((16,), lambda i: (0,)),
        ],
        out_specs=pl.BlockSpec((BLK, 16), lambda i: (i, 0)),
        out_shape=jax.ShapeDtypeStruct((NP, 16), jnp.float32),
    )(x, R0w, R0b, R1w, R1b, R2w, R2b)


def _sc_body(src_hbm, dst_hbm, lp_hbm, hw_hbm, hb_hbm,
             x_hbm, steps_hbm, rem_hbm, pa_hbm,
             sbuf, dbuf, dbufd, dbuf16d, dlowd, rows, abuf, bbuf, cbuf,
             degb5, ones, dinvb, dscaleb, sumhb, contb, stepsb, rembuf,
             dotb, pbuf, ocontb, hwbuf, hbbuf,
             spdeg, agg_sh, gsem):
    cid = lax.axis_index("c")
    sid = lax.axis_index("s")

    @pl.when(cid == 0)
    def _work():
        nbase = sid * ROWS_PER_TILE
        ebase = sid * EDGES_PER_TILE
        ngroups = ROWS_PER_TILE // 16
        zero16 = jnp.zeros((16,), jnp.float32)
        one16 = jnp.full((16,), 1.0, jnp.float32)

        def _zero_rows():
            @pl.loop(0, EW)
            def _zr(i):
                for k in range(8):
                    rows[i, pl.ds(k * 16, 16)] = zero16

        _zero_rows()

        @pl.loop(0, 16)
        def _z16(i):
            for k in range(8):
                ones[i, pl.ds(k * 16, 16)] = zero16

        @pl.loop(0, ngroups)
        def _st(i):
            sl = pl.ds(i * 16, 16)
            sumhb[sl] = zero16
            contb[sl] = one16
            stepsb[sl] = one16

        pltpu.sync_copy(hw_hbm, hwbuf)
        pltpu.sync_copy(hb_hbm, hbbuf)

        pltpu.sync_copy(rows.at[pl.ds(0, 5)], spdeg.at[pl.ds(sid * 5, 5)])
        plsc.subcore_barrier()

        @pl.loop(0, EDGES_PER_TILE // 16)
        def _deg(w):
            eb = ebase + w * 16
            pltpu.sync_copy(dst_hbm.at[pl.ds(eb, 16)], dbufd)
            v = dbufd[...]
            dbuf16d[...] = lax.shift_right_logical(v, 7)
            dlowd[...] = lax.bitwise_and(v, 127)

            @pl.loop(0, 16)
            def _oh(j):
                jj = jnp.zeros((16,), jnp.int32) + j
                lw = plsc.load_gather(dlowd, [jj])
                plsc.store_scatter(ones, [jj, lw], one16)

            pltpu.sync_copy(ones, spdeg.at[dbuf16d], add=True)

            @pl.loop(0, 16)
            def _cl(j):
                jj = jnp.zeros((16,), jnp.int32) + j
                lw = plsc.load_gather(dlowd, [jj])
                plsc.store_scatter(ones, [jj, lw], zero16)

        plsc.subcore_barrier()

        pltpu.sync_copy(spdeg.at[pl.ds(sid * 5, 5)], degb5.at[pl.ds(0, 5)])

        @pl.loop(0, 5)
        def _dinv(q):
            for k in range(8):
                deg = jnp.maximum(degb5[q, pl.ds(k * 16, 16)], 1.0)
                ii = lax.bitcast_convert_type(deg, jnp.int32)
                ii = jnp.int32(0x5F3759DF) - lax.shift_right_arithmetic(ii, 1)
                y = lax.bitcast_convert_type(ii, jnp.float32)
                half = deg * 0.5
                for _ in range(4):
                    y = y * (1.5 - half * y * y)
                sl = pl.ds(q * 128 + k * 16, 16)
                dinvb[sl] = y
                dscaleb[sl] = deg * y

        for c in range(CHUNKS):
            rb = nbase + c * NC
            sl_rows = pl.ds(rb, NC)
            pltpu.sync_copy(lp_hbm.at[sl_rows], bbuf)

            @pl.loop(0, NC)
            def _scale(i, _c=c):
                bidx = jnp.zeros((16,), jnp.int32) + (_c * NC + i)
                dv = plsc.load_gather(dinvb, [bidx])
                for k in range(8):
                    sl = pl.ds(k * 16, 16)
                    bbuf[i, sl] = bbuf[i, sl] * dv

            pltpu.sync_copy(bbuf, pa_hbm.at[sl_rows])
            pltpu.sync_copy(rows, x_hbm.at[sl_rows])

        plsc.subcore_barrier()

        hw = [hwbuf[pl.ds(k * 16, 16)] for k in range(8)]
        hb16 = hbbuf[...]

        @pl.loop(0, N_ITER)
        def _iter(t):
            _zero_rows()
            for c in range(CHUNKS):
                pltpu.sync_copy(rows, agg_sh.at[pl.ds(nbase + c * NC, NC)])
            plsc.subcore_barrier()

            @pl.loop(0, WINDOWS)
            def _prop(w):
                eb = ebase + w * EW
                pltpu.sync_copy(src_hbm.at[pl.ds(eb, EW)], sbuf)
                pltpu.sync_copy(dst_hbm.at[pl.ds(eb, EW)], dbuf)
                pltpu.async_copy(pa_hbm.at[sbuf], rows, gsem).wait()
                pltpu.sync_copy(rows, agg_sh.at[dbuf], add=True)

            plsc.subcore_barrier()

            for c in range(CHUNKS):
                rb = nbase + c * NC
                sl_rows = pl.ds(rb, NC)
                pltpu.sync_copy(agg_sh.at[sl_rows], abuf)
                pltpu.sync_copy(pa_hbm.at[sl_rows], bbuf)
                pltpu.sync_copy(x_hbm.at[sl_rows], cbuf)

                @pl.loop(0, NC)
                def _p1(i, _c=c):
                    bidx = jnp.zeros((16,), jnp.int32) + (_c * NC + i)
                    dv = plsc.load_gather(dinvb, [bidx])
                    acc = zero16
                    for k in range(8):
                        sl = pl.ds(k * 16, 16)
                        a = abuf[i, sl] * dv
                        abuf[i, sl] = a
                        acc = acc + a * hw[k]
                    plsc.store_scatter(
                        dotb, [jnp.zeros((16,), jnp.int32) + i],
                        one16 * jnp.sum(acc))

                @pl.loop(0, NC // 16)
                def _hc(gidx, _c=c):
                    lsl = pl.ds(gidx * 16, 16)
                    gsl = pl.ds(_c * NC + gidx * 16, 16)
                    d16 = dotb[lsl]
                    z = d16 + hb16
                    hh = 1.0 / (1.0 + jnp.exp(-z))
                    sh = sumhb[gsl]
                    ct = contb[gsl]
                    st = stepsb[gsl]
                    pm = jnp.where(sh + hh < 0.99, 1.0, 0.0) * ct
                    st = st + pm
                    sh = sh + pm * hh
                    fi = jnp.where(st < float(N_ITER), 1.0, 0.0)
                    cond = pm * fi
                    p = jnp.where(cond > 0.0, sh, 1.0 - sh)
                    stepsb[gsl] = st
                    sumhb[gsl] = sh
                    contb[gsl] = ct * pm
                    pbuf[lsl] = p
                    ocontb[lsl] = ct

                @pl.loop(0, NC)
                def _p2(i, _c=c):
                    gidx = jnp.zeros((16,), jnp.int32) + (_c * NC + i)
                    lidx = jnp.zeros((16,), jnp.int32) + i
                    dv = plsc.load_gather(dinvb, [gidx])
                    dsc = plsc.load_gather(dscaleb, [gidx])
                    p_s = plsc.load_gather(pbuf, [lidx])
                    sh_s = plsc.load_gather(sumhb, [gidx])
                    ct_s = plsc.load_gather(ocontb, [lidx])
                    for k in range(8):
                        sl = pl.ds(k * 16, 16)
                        new = abuf[i, sl]
                        old = bbuf[i, sl] * dsc
                        cbuf[i, sl] = cbuf[i, sl] + (new * p_s - old * sh_s) * ct_s
                        bbuf[i, sl] = new * dv

                pltpu.sync_copy(bbuf, pa_hbm.at[sl_rows])
                pltpu.sync_copy(cbuf, x_hbm.at[sl_rows])

            plsc.subcore_barrier()

        @pl.loop(0, ngroups)
        def _rem(i):
            sl = pl.ds(i * 16, 16)
            rembuf[sl] = 1.0 - sumhb[sl]

        pltpu.sync_copy(stepsb, steps_hbm.at[pl.ds(nbase, ROWS_PER_TILE)])
        pltpu.sync_copy(rembuf, rem_hbm.at[pl.ds(nbase, ROWS_PER_TILE)])


def _sc_prop(src_all, dst_all, local_preds, hw, hb16):
    mesh = plsc.VectorSubcoreMesh(core_axis_name="c", subcore_axis_name="s")
    f32 = jnp.float32
    cp = pltpu.CompilerParams()
    if "needs_layout_passes" in pltpu.CompilerParams.__dataclass_fields__:
        cp = dataclasses.replace(cp, needs_layout_passes=False)
    kern = pl.kernel(
        _sc_body,
        compiler_params=cp,
        out_type=[
            jax.ShapeDtypeStruct((NP, D), f32),
            jax.ShapeDtypeStruct((NP,), f32),
            jax.ShapeDtypeStruct((NP,), f32),
            jax.ShapeDtypeStruct((NP, D), f32),
        ],
        mesh=mesh,
        scratch_types=[
            pltpu.VMEM((EW,), jnp.int32),
            pltpu.VMEM((EW,), jnp.int32),
            pltpu.VMEM((16,), jnp.int32),
            pltpu.VMEM((16,), jnp.int32),
            pltpu.VMEM((16,), jnp.int32),
            pltpu.VMEM((EW, D), f32),
            pltpu.VMEM((NC, D), f32),
            pltpu.VMEM((NC, D), f32),
            pltpu.VMEM((NC, D), f32),
            pltpu.VMEM((8, D), f32),
            pltpu.VMEM((16, D), f32),
            pltpu.VMEM((ROWS_PER_TILE,), f32),
            pltpu.VMEM((ROWS_PER_TILE,), f32),
            pltpu.VMEM((ROWS_PER_TILE,), f32),
            pltpu.VMEM((ROWS_PER_TILE,), f32),
            pltpu.VMEM((ROWS_PER_TILE,), f32),
            pltpu.VMEM((ROWS_PER_TILE,), f32),
            pltpu.VMEM((NC,), f32),
            pltpu.VMEM((NC,), f32),
            pltpu.VMEM((NC,), f32),
            pltpu.VMEM((D,), f32),
            pltpu.VMEM((16,), f32),
            pltpu.VMEM_SHARED((NP // 128, D), f32),
            pltpu.VMEM_SHARED((NP, D), f32),
            pltpu.SemaphoreType.DMA,
        ],
    )
    return kern(src_all, dst_all, local_preds, hw, hb16)


def kernel(g, h, e, snorm_n, snorm_e, emb, W1, b1, W2, b2, W3, b3,
           halt_w, halt_b, R0w, R0b, R1w, R1b, R2w, R2b):
    loop_idx = jnp.arange(N, dtype=jnp.int32)
    pad_idx = jnp.full((PAD_EDGES,), NP - 1, jnp.int32)
    src_all = jnp.concatenate([g[0], loop_idx, pad_idx])
    dst_all = jnp.concatenate([g[1], loop_idx, pad_idx])
    h_pad = jnp.concatenate([h, jnp.zeros((NP - N,), jnp.int32)])
    h_pad = h_pad.reshape(NP // BLK, 1, BLK)
    hw = halt_w.reshape(D)
    hb16 = jnp.full((16,), halt_b[0], jnp.float32)

    local_preds = _tc_pre(h_pad, emb, W1, b1, W2, b2, W3, b3)
    x, steps, rem, _ = _sc_prop(src_all, dst_all, local_preds, hw, hb16)
    y = _tc_readout(x, R0w, R0b, R1w, R1b, R2w, R2b)
    return y[:N], steps[:N], rem[:N]


# speedup vs baseline: 2.0073x; 1.4075x over previous
import dataclasses
import functools

import jax
import jax.numpy as jnp
from jax import lax
from jax.experimental import pallas as pl
from jax.experimental.pallas import tpu as pltpu
from jax.experimental.pallas import tpu_sc as plsc

N = 10000
NP = 10240
D = 128
E_REAL = 320000
E_ALL = E_REAL + N
EPAD = 331776
PAD_EDGES = EPAD - E_ALL
ROWS_PER_TILE = NP // 16
EW = 64
NC = 64
CHUNKS = ROWS_PER_TILE // NC
EDGES_PER_TILE = EPAD // 16
WINDOWS = EDGES_PER_TILE // EW
N_ITER = 10
BLK = 256


def _tc_pre_body(h_ref, emb_ref, w1_ref, b1_ref, w2_ref, b2_ref, w3_ref,
                 b3_ref, o_ref):
    hv = h_ref[0, 0, :]
    ids = lax.broadcasted_iota(jnp.int32, (BLK, D), 1)
    oh = (hv[:, None] == ids).astype(jnp.float32)
    x = jnp.dot(oh, emb_ref[...], preferred_element_type=jnp.float32)
    x = jnp.maximum(
        jnp.dot(x, w1_ref[...], preferred_element_type=jnp.float32)
        + b1_ref[...][None, :], 0.0)
    x = jnp.maximum(
        jnp.dot(x, w2_ref[...], preferred_element_type=jnp.float32)
        + b2_ref[...][None, :], 0.0)
    x = (jnp.dot(x, w3_ref[...], preferred_element_type=jnp.float32)
         + b3_ref[...][None, :])
    o_ref[...] = x


def _tc_pre(h_pad, emb, W1, b1, W2, b2, W3, b3):
    grid = (NP // BLK,)
    full_w = pl.BlockSpec((D, D), lambda i: (0, 0))
    full_b = pl.BlockSpec((D,), lambda i: (0,))
    return pl.pallas_call(
        _tc_pre_body,
        grid=grid,
        in_specs=[
            pl.BlockSpec((1, 1, BLK), lambda i: (i, 0, 0)),
            full_w, full_w, full_b, full_w, full_b, full_w, full_b,
        ],
        out_specs=pl.BlockSpec((BLK, D), lambda i: (i, 0)),
        out_shape=jax.ShapeDtypeStruct((NP, D), jnp.float32),
    )(h_pad, emb, W1, b1, W2, b2, W3, b3)


def _tc_readout_body(x_ref, r0w_ref, r0b_ref, r1w_ref, r1b_ref, r2w_ref,
                     r2b_ref, o_ref):
    y = jnp.maximum(
        jnp.dot(x_ref[...], r0w_ref[...], preferred_element_type=jnp.float32)
        + r0b_ref[...][None, :], 0.0)
    y = jnp.maximum(
        jnp.dot(y, r1w_ref[...], preferred_element_type=jnp.float32)
        + r1b_ref[...][None, :], 0.0)
    y = (jnp.dot(y, r2w_ref[...], preferred_element_type=jnp.float32)
         + r2b_ref[...][None, :])
    o_ref[...] = y


def _tc_readout(x, R0w, R0b, R1w, R1b, R2w, R2b):
    grid = (NP // BLK,)
    return pl.pallas_call(
        _tc_readout_body,
        grid=grid,
        in_specs=[
            pl.BlockSpec((BLK, D), lambda i: (i, 0)),
            pl.BlockSpec((D, 64), lambda i: (0, 0)),
            pl.BlockSpec((64,), lambda i: (0,)),
            pl.BlockSpec((64, 32), lambda i: (0, 0)),
            pl.BlockSpec((32,), lambda i: (0,)),
            pl.BlockSpec((32, 16), lambda i: (0, 0)),
            pl.BlockSpec((16,), lambda i: (0,)),
        ],
        out_specs=pl.BlockSpec((BLK, 16), lambda i: (i, 0)),
        out_shape=jax.ShapeDtypeStruct((NP, 16), jnp.float32),
    )(x, R0w, R0b, R1w, R1b, R2w, R2b)


def _sc_body(src_hbm, dst_hbm, lp_hbm, hw_hbm, hb_hbm,
             x_hbm, steps_hbm, rem_hbm, pa_hbm,
             sbuf, dbuf, sbufB, dbufB1, dbuf16d, dlowd, rows, abuf, bbuf, cbuf,
             degb5, ones, dinvb, dscaleb, sumhb, contb, stepsb, rembuf,
             dotb, pbuf, ocontb, hwbuf, hbbuf,
             spdeg, agg_sh, gsem):
    cid = lax.axis_index("c")
    sid = lax.axis_index("s")

    @pl.when(cid == 0)
    def _work():
        nbase = sid * ROWS_PER_TILE
        ebase = sid * EDGES_PER_TILE
        ngroups = ROWS_PER_TILE // 16
        zero16 = jnp.zeros((16,), jnp.float32)
        one16 = jnp.full((16,), 1.0, jnp.float32)

        def _zero_rows():
            @pl.loop(0, EW)
            def _zr(i):
                for k in range(8):
                    rows[i, pl.ds(k * 16, 16)] = zero16

        _zero_rows()

        @pl.loop(0, 16)
        def _z16(i):
            for k in range(8):
                ones[i, pl.ds(k * 16, 16)] = zero16

        @pl.loop(0, ngroups)
        def _st(i):
            sl = pl.ds(i * 16, 16)
            sumhb[sl] = zero16
            contb[sl] = one16
            stepsb[sl] = one16

        pltpu.sync_copy(hw_hbm, hwbuf)
        pltpu.sync_copy(hb_hbm, hbbuf)

        pltpu.sync_copy(rows.at[pl.ds(0, 5)], spdeg.at[pl.ds(sid * 5, 5)])
        plsc.subcore_barrier()

        @pl.loop(0, EDGES_PER_TILE // 768)
        def _degg(g):
            geb = ebase + g * 768
            pltpu.sync_copy(dst_hbm.at[pl.ds(geb, 768)], dbufB1)
            for w in range(48):
                v = dbufB1[pl.ds(w * 16, 16)]
                dbuf16d[...] = lax.shift_right_logical(v, 7)
                dlowd[...] = lax.bitwise_and(v, 127)

                @pl.loop(0, 16)
                def _oh(j):
                    jj = jnp.zeros((16,), jnp.int32) + j
                    lw = plsc.load_gather(dlowd, [jj])
                    plsc.store_scatter(ones, [jj, lw], one16)

                pltpu.sync_copy(ones, spdeg.at[dbuf16d], add=True)

                @pl.loop(0, 16)
                def _cl(j):
                    jj = jnp.zeros((16,), jnp.int32) + j
                    lw = plsc.load_gather(dlowd, [jj])
                    plsc.store_scatter(ones, [jj, lw], zero16)

        plsc.subcore_barrier()

        pltpu.sync_copy(spdeg.at[pl.ds(sid * 5, 5)], degb5.at[pl.ds(0, 5)])

        @pl.loop(0, 5)
        def _dinv(q):
            for k in range(8):
                deg = jnp.maximum(degb5[q, pl.ds(k * 16, 16)], 1.0)
                ii = lax.bitcast_convert_type(deg, jnp.int32)
                ii = jnp.int32(0x5F3759DF) - lax.shift_right_arithmetic(ii, 1)
                y = lax.bitcast_convert_type(ii, jnp.float32)
                half = deg * 0.5
                for _ in range(4):
                    y = y * (1.5 - half * y * y)
                sl = pl.ds(q * 128 + k * 16, 16)
                dinvb[sl] = y
                dscaleb[sl] = deg * y

        for c in range(CHUNKS):
            rb = nbase + c * NC
            sl_rows = pl.ds(rb, NC)
            pltpu.sync_copy(lp_hbm.at[sl_rows], bbuf)

            @pl.loop(0, NC)
            def _scale(i, _c=c):
                bidx = jnp.zeros((16,), jnp.int32) + (_c * NC + i)
                dv = plsc.load_gather(dinvb, [bidx])
                for k in range(8):
                    sl = pl.ds(k * 16, 16)
                    bbuf[i, sl] = bbuf[i, sl] * dv

            pltpu.sync_copy(bbuf, pa_hbm.at[sl_rows])
            pltpu.sync_copy(rows, x_hbm.at[sl_rows])

        plsc.subcore_barrier()

        hw = [hwbuf[pl.ds(k * 16, 16)] for k in range(8)]
        hb16 = hbbuf[...]

        @pl.loop(0, N_ITER)
        def _iter(t):
            _zero_rows()
            for c in range(CHUNKS):
                pltpu.sync_copy(rows, agg_sh.at[pl.ds(nbase + c * NC, NC)])
            plsc.subcore_barrier()

            @pl.loop(0, WINDOWS // 12)
            def _prop(g):
                geb = ebase + g * 768
                pltpu.sync_copy(src_hbm.at[pl.ds(geb, 768)], sbufB)
                pltpu.sync_copy(dst_hbm.at[pl.ds(geb, 768)], dbufB1)
                for k in range(12):
                    for q in range(4):
                        sl = pl.ds(q * 16, 16)
                        sbuf[sl] = sbufB[pl.ds(k * 64 + q * 16, 16)]
                        dbuf[sl] = dbufB1[pl.ds(k * 64 + q * 16, 16)]
                    pltpu.async_copy(pa_hbm.at[sbuf], rows, gsem).wait()
                    pltpu.sync_copy(rows, agg_sh.at[dbuf], add=True)

            plsc.subcore_barrier()

            for c in range(CHUNKS):
                rb = nbase + c * NC
                sl_rows = pl.ds(rb, NC)
                pltpu.sync_copy(agg_sh.at[sl_rows], abuf)
                pltpu.sync_copy(pa_hbm.at[sl_rows], bbuf)
                pltpu.sync_copy(x_hbm.at[sl_rows], cbuf)

                @pl.loop(0, NC)
                def _p1(i, _c=c):
                    bidx = jnp.zeros((16,), jnp.int32) + (_c * NC + i)
                    dv = plsc.load_gather(dinvb, [bidx])
                    acc = zero16
                    for k in range(8):
                        sl = pl.ds(k * 16, 16)
                        a = abuf[i, sl] * dv
                        abuf[i, sl] = a
                        acc = acc + a * hw[k]
                    plsc.store_scatter(
                        dotb, [jnp.zeros((16,), jnp.int32) + i],
                        one16 * jnp.sum(acc))

                @pl.loop(0, NC // 16)
                def _hc(gidx, _c=c):
                    lsl = pl.ds(gidx * 16, 16)
                    gsl = pl.ds(_c * NC + gidx * 16, 16)
                    d16 = dotb[lsl]
                    z = d16 + hb16
                    hh = 1.0 / (1.0 + jnp.exp(-z))
                    sh = sumhb[gsl]
                    ct = contb[gsl]
                    st = stepsb[gsl]
                    pm = jnp.where(sh + hh < 0.99, 1.0, 0.0) * ct
                    st = st + pm
                    sh = sh + pm * hh
                    fi = jnp.where(st < float(N_ITER), 1.0, 0.0)
                    cond = pm * fi
                    p = jnp.where(cond > 0.0, sh, 1.0 - sh)
                    stepsb[gsl] = st
                    sumhb[gsl] = sh
                    contb[gsl] = ct * pm
                    pbuf[lsl] = p
                    ocontb[lsl] = ct

                @pl.loop(0, NC)
                def _p2(i, _c=c):
                    gidx = jnp.zeros((16,), jnp.int32) + (_c * NC + i)
                    lidx = jnp.zeros((16,), jnp.int32) + i
                    dv = plsc.load_gather(dinvb, [gidx])
                    dsc = plsc.load_gather(dscaleb, [gidx])
                    p_s = plsc.load_gather(pbuf, [lidx])
                    sh_s = plsc.load_gather(sumhb, [gidx])
                    ct_s = plsc.load_gather(ocontb, [lidx])
                    for k in range(8):
                        sl = pl.ds(k * 16, 16)
                        new = abuf[i, sl]
                        old = bbuf[i, sl] * dsc
                        cbuf[i, sl] = cbuf[i, sl] + (new * p_s - old * sh_s) * ct_s
                        bbuf[i, sl] = new * dv

                pltpu.sync_copy(bbuf, pa_hbm.at[sl_rows])
                pltpu.sync_copy(cbuf, x_hbm.at[sl_rows])

            plsc.subcore_barrier()

        @pl.loop(0, ngroups)
        def _rem(i):
            sl = pl.ds(i * 16, 16)
            rembuf[sl] = 1.0 - sumhb[sl]

        pltpu.sync_copy(stepsb, steps_hbm.at[pl.ds(nbase, ROWS_PER_TILE)])
        pltpu.sync_copy(rembuf, rem_hbm.at[pl.ds(nbase, ROWS_PER_TILE)])


def _sc_prop(src_all, dst_all, local_preds, hw, hb16):
    mesh = plsc.VectorSubcoreMesh(core_axis_name="c", subcore_axis_name="s")
    f32 = jnp.float32
    cp = pltpu.CompilerParams()
    if "needs_layout_passes" in pltpu.CompilerParams.__dataclass_fields__:
        cp = dataclasses.replace(cp, needs_layout_passes=False)
    kern = pl.kernel(
        _sc_body,
        compiler_params=cp,
        out_type=[
            jax.ShapeDtypeStruct((NP, D), f32),
            jax.ShapeDtypeStruct((NP,), f32),
            jax.ShapeDtypeStruct((NP,), f32),
            jax.ShapeDtypeStruct((NP, D), f32),
        ],
        mesh=mesh,
        scratch_types=[
            pltpu.VMEM((EW,), jnp.int32),
            pltpu.VMEM((EW,), jnp.int32),
            pltpu.VMEM((768,), jnp.int32),
            pltpu.VMEM((768,), jnp.int32),
            pltpu.VMEM((16,), jnp.int32),
            pltpu.VMEM((16,), jnp.int32),
            pltpu.VMEM((EW, D), f32),
            pltpu.VMEM((NC, D), f32),
            pltpu.VMEM((NC, D), f32),
            pltpu.VMEM((NC, D), f32),
            pltpu.VMEM((8, D), f32),
            pltpu.VMEM((16, D), f32),
            pltpu.VMEM((ROWS_PER_TILE,), f32),
            pltpu.VMEM((ROWS_PER_TILE,), f32),
            pltpu.VMEM((ROWS_PER_TILE,), f32),
            pltpu.VMEM((ROWS_PER_TILE,), f32),
            pltpu.VMEM((ROWS_PER_TILE,), f32),
            pltpu.VMEM((ROWS_PER_TILE,), f32),
            pltpu.VMEM((NC,), f32),
            pltpu.VMEM((NC,), f32),
            pltpu.VMEM((NC,), f32),
            pltpu.VMEM((D,), f32),
            pltpu.VMEM((16,), f32),
            pltpu.VMEM_SHARED((NP // 128, D), f32),
            pltpu.VMEM_SHARED((NP, D), f32),
            pltpu.SemaphoreType.DMA,
        ],
    )
    return kern(src_all, dst_all, local_preds, hw, hb16)


def kernel(g, h, e, snorm_n, snorm_e, emb, W1, b1, W2, b2, W3, b3,
           halt_w, halt_b, R0w, R0b, R1w, R1b, R2w, R2b):
    loop_idx = jnp.arange(N, dtype=jnp.int32)
    pad_idx = jnp.full((PAD_EDGES,), NP - 1, jnp.int32)
    src_all = jnp.concatenate([g[0], loop_idx, pad_idx])
    dst_all = jnp.concatenate([g[1], loop_idx, pad_idx])
    h_pad = jnp.concatenate([h, jnp.zeros((NP - N,), jnp.int32)])
    h_pad = h_pad.reshape(NP // BLK, 1, BLK)
    hw = halt_w.reshape(D)
    hb16 = jnp.full((16,), halt_b[0], jnp.float32)

    local_preds = _tc_pre(h_pad, emb, W1, b1, W2, b2, W3, b3)
    x, steps, rem, _ = _sc_prop(src_all, dst_all, local_preds, hw, hb16)
    y = _tc_readout(x, R0w, R0b, R1w, R1b, R2w, R2b)
    return y[:N], steps[:N], rem[:N]


# speedup vs baseline: 2.3652x; 1.1783x over previous
import dataclasses
import functools

import jax
import jax.numpy as jnp
from jax import lax
from jax.experimental import pallas as pl
from jax.experimental.pallas import tpu as pltpu
from jax.experimental.pallas import tpu_sc as plsc

N = 10000
NP = 10240
D = 128
E_REAL = 320000
E_ALL = E_REAL + N
EPAD = 331776
PAD_EDGES = EPAD - E_ALL
ROWS_PER_TILE = NP // 16
EW = 128
NC = 64
CHUNKS = ROWS_PER_TILE // NC
EDGES_PER_TILE = EPAD // 16
WINDOWS = EDGES_PER_TILE // EW
N_ITER = 10
BLK = 256


def _tc_pre_body(h_ref, emb_ref, w1_ref, b1_ref, w2_ref, b2_ref, w3_ref,
                 b3_ref, o_ref):
    hv = h_ref[0, 0, :]
    ids = lax.broadcasted_iota(jnp.int32, (BLK, D), 1)
    oh = (hv[:, None] == ids).astype(jnp.float32)
    x = jnp.dot(oh, emb_ref[...], preferred_element_type=jnp.float32)
    x = jnp.maximum(
        jnp.dot(x, w1_ref[...], preferred_element_type=jnp.float32)
        + b1_ref[...][None, :], 0.0)
    x = jnp.maximum(
        jnp.dot(x, w2_ref[...], preferred_element_type=jnp.float32)
        + b2_ref[...][None, :], 0.0)
    x = (jnp.dot(x, w3_ref[...], preferred_element_type=jnp.float32)
         + b3_ref[...][None, :])
    o_ref[...] = x


def _tc_pre(h_pad, emb, W1, b1, W2, b2, W3, b3):
    grid = (NP // BLK,)
    full_w = pl.BlockSpec((D, D), lambda i: (0, 0))
    full_b = pl.BlockSpec((D,), lambda i: (0,))
    return pl.pallas_call(
        _tc_pre_body,
        grid=grid,
        in_specs=[
            pl.BlockSpec((1, 1, BLK), lambda i: (i, 0, 0)),
            full_w, full_w, full_b, full_w, full_b, full_w, full_b,
        ],
        out_specs=pl.BlockSpec((BLK, D), lambda i: (i, 0)),
        out_shape=jax.ShapeDtypeStruct((NP, D), jnp.float32),
    )(h_pad, emb, W1, b1, W2, b2, W3, b3)


def _tc_readout_body(x_ref, r0w_ref, r0b_ref, r1w_ref, r1b_ref, r2w_ref,
                     r2b_ref, o_ref):
    y = jnp.maximum(
        jnp.dot(x_ref[...], r0w_ref[...], preferred_element_type=jnp.float32)
        + r0b_ref[...][None, :], 0.0)
    y = jnp.maximum(
        jnp.dot(y, r1w_ref[...], preferred_element_type=jnp.float32)
        + r1b_ref[...][None, :], 0.0)
    y = (jnp.dot(y, r2w_ref[...], preferred_element_type=jnp.float32)
         + r2b_ref[...][None, :])
    o_ref[...] = y


def _tc_readout(x, R0w, R0b, R1w, R1b, R2w, R2b):
    grid = (NP // BLK,)
    return pl.pallas_call(
        _tc_readout_body,
        grid=grid,
        in_specs=[
            pl.BlockSpec((BLK, D), lambda i: (i, 0)),
            pl.BlockSpec((D, 64), lambda i: (0, 0)),
            pl.BlockSpec((64,), lambda i: (0,)),
            pl.BlockSpec((64, 32), lambda i: (0, 0)),
            pl.BlockSpec((32,), lambda i: (0,)),
            pl.BlockSpec((32, 16), lambda i: (0, 0)),
            pl.BlockSpec((16,), lambda i: (0,)),
        ],
        out_specs=pl.BlockSpec((BLK, 16), lambda i: (i, 0)),
        out_shape=jax.ShapeDtypeStruct((NP, 16), jnp.float32),
    )(x, R0w, R0b, R1w, R1b, R2w, R2b)


def _sc_body(src_hbm, dst_hbm, lp_hbm, hw_hbm, hb_hbm,
             x_hbm, steps_hbm, rem_hbm, pa_hbm,
             sbuf, dbuf, sbufB, dbufB1, dbuf16d, dlowd, rows, abuf, bbuf,
             degb5, ones, dinvb, dscaleb, sumhb, contb, stepsb, rembuf,
             dotb, pbuf, ocontb, hwbuf, hbbuf,
             spdeg, agg_sh, gsem):
    cid = lax.axis_index("c")
    sid = lax.axis_index("s")

    @pl.when(cid == 0)
    def _work():
        nbase = sid * ROWS_PER_TILE
        ebase = sid * EDGES_PER_TILE
        ngroups = ROWS_PER_TILE // 16
        zero16 = jnp.zeros((16,), jnp.float32)
        one16 = jnp.full((16,), 1.0, jnp.float32)

        def _zero_rows():
            @pl.loop(0, EW)
            def _zr(i):
                for k in range(8):
                    rows[i, pl.ds(k * 16, 16)] = zero16

        _zero_rows()

        @pl.loop(0, 16)
        def _z16(i):
            for k in range(8):
                ones[i, pl.ds(k * 16, 16)] = zero16

        @pl.loop(0, ngroups)
        def _st(i):
            sl = pl.ds(i * 16, 16)
            sumhb[sl] = zero16
            contb[sl] = one16
            stepsb[sl] = one16

        pltpu.sync_copy(hw_hbm, hwbuf)
        pltpu.sync_copy(hb_hbm, hbbuf)

        pltpu.sync_copy(rows.at[pl.ds(0, 5)], spdeg.at[pl.ds(sid * 5, 5)])
        plsc.subcore_barrier()

        @pl.loop(0, EDGES_PER_TILE // 768)
        def _degg(g):
            geb = ebase + g * 768
            pltpu.sync_copy(dst_hbm.at[pl.ds(geb, 768)], dbufB1)
            for w in range(48):
                v = dbufB1[pl.ds(w * 16, 16)]
                dbuf16d[...] = lax.shift_right_logical(v, 7)
                dlowd[...] = lax.bitwise_and(v, 127)

                @pl.loop(0, 16)
                def _oh(j):
                    jj = jnp.zeros((16,), jnp.int32) + j
                    lw = plsc.load_gather(dlowd, [jj])
                    plsc.store_scatter(ones, [jj, lw], one16)

                pltpu.sync_copy(ones, spdeg.at[dbuf16d], add=True)

                @pl.loop(0, 16)
                def _cl(j):
                    jj = jnp.zeros((16,), jnp.int32) + j
                    lw = plsc.load_gather(dlowd, [jj])
                    plsc.store_scatter(ones, [jj, lw], zero16)

        plsc.subcore_barrier()

        pltpu.sync_copy(spdeg.at[pl.ds(sid * 5, 5)], degb5.at[pl.ds(0, 5)])

        @pl.loop(0, 5)
        def _dinv(q):
            for k in range(8):
                deg = jnp.maximum(degb5[q, pl.ds(k * 16, 16)], 1.0)
                ii = lax.bitcast_convert_type(deg, jnp.int32)
                ii = jnp.int32(0x5F3759DF) - lax.shift_right_arithmetic(ii, 1)
                y = lax.bitcast_convert_type(ii, jnp.float32)
                half = deg * 0.5
                for _ in range(4):
                    y = y * (1.5 - half * y * y)
                sl = pl.ds(q * 128 + k * 16, 16)
                dinvb[sl] = y
                dscaleb[sl] = deg * y

        for c in range(CHUNKS):
            rb = nbase + c * NC
            sl_rows = pl.ds(rb, NC)
            pltpu.sync_copy(lp_hbm.at[sl_rows], bbuf)

            @pl.loop(0, NC)
            def _scale(i, _c=c):
                bidx = jnp.zeros((16,), jnp.int32) + (_c * NC + i)
                dv = plsc.load_gather(dinvb, [bidx])
                for k in range(8):
                    sl = pl.ds(k * 16, 16)
                    bbuf[i, sl] = bbuf[i, sl] * dv

            pltpu.sync_copy(bbuf, pa_hbm.at[sl_rows])
            pltpu.sync_copy(rows.at[pl.ds(0, NC)], x_hbm.at[sl_rows])

        plsc.subcore_barrier()

        hw = [hwbuf[pl.ds(k * 16, 16)] for k in range(8)]
        hb16 = hbbuf[...]

        @pl.loop(0, N_ITER)
        def _iter(t):
            _zero_rows()
            for c in range(5):
                pltpu.sync_copy(rows, agg_sh.at[pl.ds(nbase + c * EW, EW)])
            plsc.subcore_barrier()

            @pl.loop(0, WINDOWS // 6)
            def _prop(g):
                geb = ebase + g * 768
                pltpu.sync_copy(src_hbm.at[pl.ds(geb, 768)], sbufB)
                pltpu.sync_copy(dst_hbm.at[pl.ds(geb, 768)], dbufB1)
                for k in range(6):
                    for q in range(8):
                        sl = pl.ds(q * 16, 16)
                        sbuf[sl] = sbufB[pl.ds(k * 128 + q * 16, 16)]
                        dbuf[sl] = dbufB1[pl.ds(k * 128 + q * 16, 16)]
                    pltpu.async_copy(pa_hbm.at[sbuf], rows, gsem).wait()
                    pltpu.sync_copy(rows, agg_sh.at[dbuf], add=True)

            plsc.subcore_barrier()

            for c in range(CHUNKS):
                rb = nbase + c * NC
                sl_rows = pl.ds(rb, NC)
                pltpu.sync_copy(agg_sh.at[sl_rows], abuf)
                pltpu.sync_copy(pa_hbm.at[sl_rows], bbuf)
                pltpu.sync_copy(x_hbm.at[sl_rows], rows.at[pl.ds(0, NC)])

                @pl.loop(0, NC)
                def _p1(i, _c=c):
                    bidx = jnp.zeros((16,), jnp.int32) + (_c * NC + i)
                    dv = plsc.load_gather(dinvb, [bidx])
                    acc = zero16
                    for k in range(8):
                        sl = pl.ds(k * 16, 16)
                        a = abuf[i, sl] * dv
                        abuf[i, sl] = a
                        acc = acc + a * hw[k]
                    plsc.store_scatter(
                        dotb, [jnp.zeros((16,), jnp.int32) + i],
                        one16 * jnp.sum(acc))

                @pl.loop(0, NC // 16)
                def _hc(gidx, _c=c):
                    lsl = pl.ds(gidx * 16, 16)
                    gsl = pl.ds(_c * NC + gidx * 16, 16)
                    d16 = dotb[lsl]
                    z = d16 + hb16
                    hh = 1.0 / (1.0 + jnp.exp(-z))
                    sh = sumhb[gsl]
                    ct = contb[gsl]
                    st = stepsb[gsl]
                    pm = jnp.where(sh + hh < 0.99, 1.0, 0.0) * ct
                    st = st + pm
                    sh = sh + pm * hh
                    fi = jnp.where(st < float(N_ITER), 1.0, 0.0)
                    cond = pm * fi
                    p = jnp.where(cond > 0.0, sh, 1.0 - sh)
                    stepsb[gsl] = st
                    sumhb[gsl] = sh
                    contb[gsl] = ct * pm
                    pbuf[lsl] = p
                    ocontb[lsl] = ct

                @pl.loop(0, NC)
                def _p2(i, _c=c):
                    gidx = jnp.zeros((16,), jnp.int32) + (_c * NC + i)
                    lidx = jnp.zeros((16,), jnp.int32) + i
                    dv = plsc.load_gather(dinvb, [gidx])
                    dsc = plsc.load_gather(dscaleb, [gidx])
                    p_s = plsc.load_gather(pbuf, [lidx])
                    sh_s = plsc.load_gather(sumhb, [gidx])
                    ct_s = plsc.load_gather(ocontb, [lidx])
                    for k in range(8):
                        sl = pl.ds(k * 16, 16)
                        new = abuf[i, sl]
                        old = bbuf[i, sl] * dsc
                        rows[i, sl] = rows[i, sl] + (new * p_s - old * sh_s) * ct_s
                        bbuf[i, sl] = new * dv

                pltpu.sync_copy(bbuf, pa_hbm.at[sl_rows])
                pltpu.sync_copy(rows.at[pl.ds(0, NC)], x_hbm.at[sl_rows])

            plsc.subcore_barrier()

        @pl.loop(0, ngroups)
        def _rem(i):
            sl = pl.ds(i * 16, 16)
            rembuf[sl] = 1.0 - sumhb[sl]

        pltpu.sync_copy(stepsb, steps_hbm.at[pl.ds(nbase, ROWS_PER_TILE)])
        pltpu.sync_copy(rembuf, rem_hbm.at[pl.ds(nbase, ROWS_PER_TILE)])


def _sc_prop(src_all, dst_all, local_preds, hw, hb16):
    mesh = plsc.VectorSubcoreMesh(core_axis_name="c", subcore_axis_name="s")
    f32 = jnp.float32
    cp = pltpu.CompilerParams()
    if "needs_layout_passes" in pltpu.CompilerParams.__dataclass_fields__:
        cp = dataclasses.replace(cp, needs_layout_passes=False)
    kern = pl.kernel(
        _sc_body,
        compiler_params=cp,
        out_type=[
            jax.ShapeDtypeStruct((NP, D), f32),
            jax.ShapeDtypeStruct((NP,), f32),
            jax.ShapeDtypeStruct((NP,), f32),
            jax.ShapeDtypeStruct((NP, D), f32),
        ],
        mesh=mesh,
        scratch_types=[
            pltpu.VMEM((EW,), jnp.int32),
            pltpu.VMEM((EW,), jnp.int32),
            pltpu.VMEM((768,), jnp.int32),
            pltpu.VMEM((768,), jnp.int32),
            pltpu.VMEM((16,), jnp.int32),
            pltpu.VMEM((16,), jnp.int32),
            pltpu.VMEM((EW, D), f32),
            pltpu.VMEM((NC, D), f32),
            pltpu.VMEM((NC, D), f32),
            pltpu.VMEM((8, D), f32),
            pltpu.VMEM((16, D), f32),
            pltpu.VMEM((ROWS_PER_TILE,), f32),
            pltpu.VMEM((ROWS_PER_TILE,), f32),
            pltpu.VMEM((ROWS_PER_TILE,), f32),
            pltpu.VMEM((ROWS_PER_TILE,), f32),
            pltpu.VMEM((ROWS_PER_TILE,), f32),
            pltpu.VMEM((ROWS_PER_TILE,), f32),
            pltpu.VMEM((NC,), f32),
            pltpu.VMEM((NC,), f32),
            pltpu.VMEM((NC,), f32),
            pltpu.VMEM((D,), f32),
            pltpu.VMEM((16,), f32),
            pltpu.VMEM_SHARED((NP // 128, D), f32),
            pltpu.VMEM_SHARED((NP, D), f32),
            pltpu.SemaphoreType.DMA,
        ],
    )
    return kern(src_all, dst_all, local_preds, hw, hb16)


def kernel(g, h, e, snorm_n, snorm_e, emb, W1, b1, W2, b2, W3, b3,
           halt_w, halt_b, R0w, R0b, R1w, R1b, R2w, R2b):
    loop_idx = jnp.arange(N, dtype=jnp.int32)
    pad_idx = jnp.full((PAD_EDGES,), NP - 1, jnp.int32)
    src_all = jnp.concatenate([g[0], loop_idx, pad_idx])
    dst_all = jnp.concatenate([g[1], loop_idx, pad_idx])
    h_pad = jnp.concatenate([h, jnp.zeros((NP - N,), jnp.int32)])
    h_pad = h_pad.reshape(NP // BLK, 1, BLK)
    hw = halt_w.reshape(D)
    hb16 = jnp.full((16,), halt_b[0], jnp.float32)

    local_preds = _tc_pre(h_pad, emb, W1, b1, W2, b2, W3, b3)
    x, steps, rem, _ = _sc_prop(src_all, dst_all, local_preds, hw, hb16)
    y = _tc_readout(x, R0w, R0b, R1w, R1b, R2w, R2b)
    return y[:N], steps[:N], rem[:N]


# speedup vs baseline: 2.3997x; 1.0146x over previous
import dataclasses
import functools

import jax
import jax.numpy as jnp
from jax import lax
from jax.experimental import pallas as pl
from jax.experimental.pallas import tpu as pltpu
from jax.experimental.pallas import tpu_sc as plsc

N = 10000
NP = 10240
D = 128
E_REAL = 320000
E_ALL = E_REAL + N
EPAD = 331776
PAD_EDGES = EPAD - E_ALL
ROWS_PER_TILE = NP // 16
EW = 128
NC = 64
CHUNKS = ROWS_PER_TILE // NC
EDGES_PER_TILE = EPAD // 16
WINDOWS = EDGES_PER_TILE // EW
N_ITER = 10
BLK = 256


def _tc_pre_body(h_ref, emb_ref, w1_ref, b1_ref, w2_ref, b2_ref, w3_ref,
                 b3_ref, o_ref):
    hv = h_ref[0, 0, :]
    ids = lax.broadcasted_iota(jnp.int32, (BLK, D), 1)
    oh = (hv[:, None] == ids).astype(jnp.float32)
    x = jnp.dot(oh, emb_ref[...], preferred_element_type=jnp.float32)
    x = jnp.maximum(
        jnp.dot(x, w1_ref[...], preferred_element_type=jnp.float32)
        + b1_ref[...][None, :], 0.0)
    x = jnp.maximum(
        jnp.dot(x, w2_ref[...], preferred_element_type=jnp.float32)
        + b2_ref[...][None, :], 0.0)
    x = (jnp.dot(x, w3_ref[...], preferred_element_type=jnp.float32)
         + b3_ref[...][None, :])
    o_ref[...] = x


def _tc_pre(h_pad, emb, W1, b1, W2, b2, W3, b3):
    grid = (NP // BLK,)
    full_w = pl.BlockSpec((D, D), lambda i: (0, 0))
    full_b = pl.BlockSpec((D,), lambda i: (0,))
    return pl.pallas_call(
        _tc_pre_body,
        grid=grid,
        in_specs=[
            pl.BlockSpec((1, 1, BLK), lambda i: (i, 0, 0)),
            full_w, full_w, full_b, full_w, full_b, full_w, full_b,
        ],
        out_specs=pl.BlockSpec((BLK, D), lambda i: (i, 0)),
        out_shape=jax.ShapeDtypeStruct((NP, D), jnp.float32),
    )(h_pad, emb, W1, b1, W2, b2, W3, b3)


def _tc_readout_body(x_ref, r0w_ref, r0b_ref, r1w_ref, r1b_ref, r2w_ref,
                     r2b_ref, o_ref):
    y = jnp.maximum(
        jnp.dot(x_ref[...], r0w_ref[...], preferred_element_type=jnp.float32)
        + r0b_ref[...][None, :], 0.0)
    y = jnp.maximum(
        jnp.dot(y, r1w_ref[...], preferred_element_type=jnp.float32)
        + r1b_ref[...][None, :], 0.0)
    y = (jnp.dot(y, r2w_ref[...], preferred_element_type=jnp.float32)
         + r2b_ref[...][None, :])
    o_ref[...] = y


def _tc_readout(x, R0w, R0b, R1w, R1b, R2w, R2b):
    grid = (NP // BLK,)
    return pl.pallas_call(
        _tc_readout_body,
        grid=grid,
        in_specs=[
            pl.BlockSpec((BLK, D), lambda i: (i, 0)),
            pl.BlockSpec((D, 64), lambda i: (0, 0)),
            pl.BlockSpec((64,), lambda i: (0,)),
            pl.BlockSpec((64, 32), lambda i: (0, 0)),
            pl.BlockSpec((32,), lambda i: (0,)),
            pl.BlockSpec((32, 16), lambda i: (0, 0)),
            pl.BlockSpec((16,), lambda i: (0,)),
        ],
        out_specs=pl.BlockSpec((BLK, 16), lambda i: (i, 0)),
        out_shape=jax.ShapeDtypeStruct((NP, 16), jnp.float32),
    )(x, R0w, R0b, R1w, R1b, R2w, R2b)


def _sc_body(src_hbm, dst_hbm, lp_hbm, hw_hbm, hb_hbm,
             x_hbm, steps_hbm, rem_hbm, pa_hbm,
             sbuf, dbuf, sbufB, dbufB1, dbuf16d, dlowd, rows, abuf, bbuf,
             degb5, ones, dinvb, dscaleb, sumhb, contb, stepsb, rembuf,
             dotb, pbuf, ocontb, hwbuf, hbbuf,
             spdeg, agg_sh, gsem):
    cid = lax.axis_index("c")
    sid = lax.axis_index("s")

    @pl.when(cid == 0)
    def _work():
        nbase = sid * ROWS_PER_TILE
        ebase = sid * EDGES_PER_TILE
        ngroups = ROWS_PER_TILE // 16
        zero16 = jnp.zeros((16,), jnp.float32)
        one16 = jnp.full((16,), 1.0, jnp.float32)

        def _zero_rows():
            @pl.loop(0, EW)
            def _zr(i):
                for k in range(8):
                    rows[i, pl.ds(k * 16, 16)] = zero16

        _zero_rows()

        @pl.loop(0, 16)
        def _z16(i):
            for k in range(8):
                ones[i, pl.ds(k * 16, 16)] = zero16

        @pl.loop(0, ngroups)
        def _st(i):
            sl = pl.ds(i * 16, 16)
            sumhb[sl] = zero16
            contb[sl] = one16
            stepsb[sl] = one16

        pltpu.sync_copy(hw_hbm, hwbuf)
        pltpu.sync_copy(hb_hbm, hbbuf)

        pltpu.sync_copy(rows.at[pl.ds(0, 5)], spdeg.at[pl.ds(sid * 5, 5)])
        plsc.subcore_barrier()

        @pl.loop(0, EDGES_PER_TILE // 768)
        def _degg(g):
            geb = ebase + g * 768
            pltpu.sync_copy(dst_hbm.at[pl.ds(geb, 768)], dbufB1)
            for w in range(6):
                for q in range(8):
                    sl = pl.ds(q * 16, 16)
                    v = dbufB1[pl.ds(w * 128 + q * 16, 16)]
                    sbuf[sl] = lax.shift_right_logical(v, 7)
                    dbuf[sl] = lax.bitwise_and(v, 127)

                @pl.loop(0, 128)
                def _oh(j):
                    jj = jnp.zeros((16,), jnp.int32) + j
                    lw = plsc.load_gather(dbuf, [jj])
                    plsc.store_scatter(rows, [jj, lw], one16)

                pltpu.sync_copy(rows, spdeg.at[sbuf], add=True)

                @pl.loop(0, 128)
                def _cl(j):
                    jj = jnp.zeros((16,), jnp.int32) + j
                    lw = plsc.load_gather(dbuf, [jj])
                    plsc.store_scatter(rows, [jj, lw], zero16)

        plsc.subcore_barrier()

        pltpu.sync_copy(spdeg.at[pl.ds(sid * 5, 5)], degb5.at[pl.ds(0, 5)])

        @pl.loop(0, 5)
        def _dinv(q):
            for k in range(8):
                deg = jnp.maximum(degb5[q, pl.ds(k * 16, 16)], 1.0)
                ii = lax.bitcast_convert_type(deg, jnp.int32)
                ii = jnp.int32(0x5F3759DF) - lax.shift_right_arithmetic(ii, 1)
                y = lax.bitcast_convert_type(ii, jnp.float32)
                half = deg * 0.5
                for _ in range(4):
                    y = y * (1.5 - half * y * y)
                sl = pl.ds(q * 128 + k * 16, 16)
                dinvb[sl] = y
                dscaleb[sl] = deg * y

        for c in range(CHUNKS):
            rb = nbase + c * NC
            sl_rows = pl.ds(rb, NC)
            pltpu.sync_copy(lp_hbm.at[sl_rows], bbuf)

            @pl.loop(0, NC)
            def _scale(i, _c=c):
                bidx = jnp.zeros((16,), jnp.int32) + (_c * NC + i)
                dv = plsc.load_gather(dinvb, [bidx])
                for k in range(8):
                    sl = pl.ds(k * 16, 16)
                    bbuf[i, sl] = bbuf[i, sl] * dv

            pltpu.sync_copy(bbuf, pa_hbm.at[sl_rows])
            pltpu.sync_copy(rows.at[pl.ds(0, NC)], x_hbm.at[sl_rows])

        plsc.subcore_barrier()

        hw = [hwbuf[pl.ds(k * 16, 16)] for k in range(8)]
        hb16 = hbbuf[...]

        @pl.loop(0, N_ITER)
        def _iter(t):
            _zero_rows()
            for c in range(5):
                pltpu.sync_copy(rows, agg_sh.at[pl.ds(nbase + c * EW, EW)])
            plsc.subcore_barrier()

            @pl.loop(0, WINDOWS // 6)
            def _prop(g):
                geb = ebase + g * 768
                pltpu.sync_copy(src_hbm.at[pl.ds(geb, 768)], sbufB)
                pltpu.sync_copy(dst_hbm.at[pl.ds(geb, 768)], dbufB1)
                for k in range(6):
                    for q in range(8):
                        sl = pl.ds(q * 16, 16)
                        sbuf[sl] = sbufB[pl.ds(k * 128 + q * 16, 16)]
                        dbuf[sl] = dbufB1[pl.ds(k * 128 + q * 16, 16)]
                    pltpu.async_copy(pa_hbm.at[sbuf], rows, gsem).wait()
                    pltpu.sync_copy(rows, agg_sh.at[dbuf], add=True)

            plsc.subcore_barrier()

            for c in range(CHUNKS):
                rb = nbase + c * NC
                sl_rows = pl.ds(rb, NC)
                pltpu.sync_copy(agg_sh.at[sl_rows], abuf)
                pltpu.sync_copy(pa_hbm.at[sl_rows], bbuf)
                pltpu.sync_copy(x_hbm.at[sl_rows], rows.at[pl.ds(0, NC)])

                @pl.loop(0, NC)
                def _p1(i, _c=c):
                    bidx = jnp.zeros((16,), jnp.int32) + (_c * NC + i)
                    dv = plsc.load_gather(dinvb, [bidx])
                    acc = zero16
                    for k in range(8):
                        sl = pl.ds(k * 16, 16)
                        a = abuf[i, sl] * dv
                        abuf[i, sl] = a
                        acc = acc + a * hw[k]
                    plsc.store_scatter(
                        dotb, [jnp.zeros((16,), jnp.int32) + i],
                        one16 * jnp.sum(acc))

                @pl.loop(0, NC // 16)
                def _hc(gidx, _c=c):
                    lsl = pl.ds(gidx * 16, 16)
                    gsl = pl.ds(_c * NC + gidx * 16, 16)
                    d16 = dotb[lsl]
                    z = d16 + hb16
                    hh = 1.0 / (1.0 + jnp.exp(-z))
                    sh = sumhb[gsl]
                    ct = contb[gsl]
                    st = stepsb[gsl]
                    pm = jnp.where(sh + hh < 0.99, 1.0, 0.0) * ct
                    st = st + pm
                    sh = sh + pm * hh
                    fi = jnp.where(st < float(N_ITER), 1.0, 0.0)
                    cond = pm * fi
                    p = jnp.where(cond > 0.0, sh, 1.0 - sh)
                    stepsb[gsl] = st
                    sumhb[gsl] = sh
                    contb[gsl] = ct * pm
                    pbuf[lsl] = p
                    ocontb[lsl] = ct

                @pl.loop(0, NC)
                def _p2(i, _c=c):
                    gidx = jnp.zeros((16,), jnp.int32) + (_c * NC + i)
                    lidx = jnp.zeros((16,), jnp.int32) + i
                    dv = plsc.load_gather(dinvb, [gidx])
                    dsc = plsc.load_gather(dscaleb, [gidx])
                    p_s = plsc.load_gather(pbuf, [lidx])
                    sh_s = plsc.load_gather(sumhb, [gidx])
                    ct_s = plsc.load_gather(ocontb, [lidx])
                    for k in range(8):
                        sl = pl.ds(k * 16, 16)
                        new = abuf[i, sl]
                        old = bbuf[i, sl] * dsc
                        rows[i, sl] = rows[i, sl] + (new * p_s - old * sh_s) * ct_s
                        bbuf[i, sl] = new * dv

                pltpu.sync_copy(bbuf, pa_hbm.at[sl_rows])
                pltpu.sync_copy(rows.at[pl.ds(0, NC)], x_hbm.at[sl_rows])

            plsc.subcore_barrier()

        @pl.loop(0, ngroups)
        def _rem(i):
            sl = pl.ds(i * 16, 16)
            rembuf[sl] = 1.0 - sumhb[sl]

        pltpu.sync_copy(stepsb, steps_hbm.at[pl.ds(nbase, ROWS_PER_TILE)])
        pltpu.sync_copy(rembuf, rem_hbm.at[pl.ds(nbase, ROWS_PER_TILE)])


def _sc_prop(src_all, dst_all, local_preds, hw, hb16):
    mesh = plsc.VectorSubcoreMesh(core_axis_name="c", subcore_axis_name="s")
    f32 = jnp.float32
    cp = pltpu.CompilerParams()
    if "needs_layout_passes" in pltpu.CompilerParams.__dataclass_fields__:
        cp = dataclasses.replace(cp, needs_layout_passes=False)
    kern = pl.kernel(
        _sc_body,
        compiler_params=cp,
        out_type=[
            jax.ShapeDtypeStruct((NP, D), f32),
            jax.ShapeDtypeStruct((NP,), f32),
            jax.ShapeDtypeStruct((NP,), f32),
            jax.ShapeDtypeStruct((NP, D), f32),
        ],
        mesh=mesh,
        scratch_types=[
            pltpu.VMEM((EW,), jnp.int32),
            pltpu.VMEM((EW,), jnp.int32),
            pltpu.VMEM((768,), jnp.int32),
            pltpu.VMEM((768,), jnp.int32),
            pltpu.VMEM((16,), jnp.int32),
            pltpu.VMEM((16,), jnp.int32),
            pltpu.VMEM((EW, D), f32),
            pltpu.VMEM((NC, D), f32),
            pltpu.VMEM((NC, D), f32),
            pltpu.VMEM((8, D), f32),
            pltpu.VMEM((16, D), f32),
            pltpu.VMEM((ROWS_PER_TILE,), f32),
            pltpu.VMEM((ROWS_PER_TILE,), f32),
            pltpu.VMEM((ROWS_PER_TILE,), f32),
            pltpu.VMEM((ROWS_PER_TILE,), f32),
            pltpu.VMEM((ROWS_PER_TILE,), f32),
            pltpu.VMEM((ROWS_PER_TILE,), f32),
            pltpu.VMEM((NC,), f32),
            pltpu.VMEM((NC,), f32),
            pltpu.VMEM((NC,), f32),
            pltpu.VMEM((D,), f32),
            pltpu.VMEM((16,), f32),
            pltpu.VMEM_SHARED((NP // 128, D), f32),
            pltpu.VMEM_SHARED((NP, D), f32),
            pltpu.SemaphoreType.DMA,
        ],
    )
    return kern(src_all, dst_all, local_preds, hw, hb16)


def kernel(g, h, e, snorm_n, snorm_e, emb, W1, b1, W2, b2, W3, b3,
           halt_w, halt_b, R0w, R0b, R1w, R1b, R2w, R2b):
    loop_idx = jnp.arange(N, dtype=jnp.int32)
    pad_idx = jnp.full((PAD_EDGES,), NP - 1, jnp.int32)
    src_all = jnp.concatenate([g[0], loop_idx, pad_idx])
    dst_all = jnp.concatenate([g[1], loop_idx, pad_idx])
    h_pad = jnp.concatenate([h, jnp.zeros((NP - N,), jnp.int32)])
    h_pad = h_pad.reshape(NP // BLK, 1, BLK)
    hw = halt_w.reshape(D)
    hb16 = jnp.full((16,), halt_b[0], jnp.float32)

    local_preds = _tc_pre(h_pad, emb, W1, b1, W2, b2, W3, b3)
    x, steps, rem, _ = _sc_prop(src_all, dst_all, local_preds, hw, hb16)
    y = _tc_readout(x, R0w, R0b, R1w, R1b, R2w, R2b)
    return y[:N], steps[:N], rem[:N]


# speedup vs baseline: 2.7878x; 1.1617x over previous
import dataclasses
import functools

import jax
import jax.numpy as jnp
from jax import lax
from jax.experimental import pallas as pl
from jax.experimental.pallas import tpu as pltpu
from jax.experimental.pallas import tpu_sc as plsc

N = 10000
NP = 10240
D = 128
E_REAL = 320000
E_ALL = E_REAL + N
EPAD = 331776
PAD_EDGES = EPAD - E_ALL
ROWS_PER_TILE = NP // 16
EW = 128
NC = 64
CHUNKS = ROWS_PER_TILE // NC
EDGES_PER_TILE = EPAD // 16
WINDOWS = EDGES_PER_TILE // EW
N_ITER = 10
BLK = 256


def _tc_pre_body(h_ref, emb_ref, w1_ref, b1_ref, w2_ref, b2_ref, w3_ref,
                 b3_ref, o_ref):
    hv = h_ref[0, 0, :]
    ids = lax.broadcasted_iota(jnp.int32, (BLK, D), 1)
    oh = (hv[:, None] == ids).astype(jnp.float32)
    x = jnp.dot(oh, emb_ref[...], preferred_element_type=jnp.float32)
    x = jnp.maximum(
        jnp.dot(x, w1_ref[...], preferred_element_type=jnp.float32)
        + b1_ref[...][None, :], 0.0)
    x = jnp.maximum(
        jnp.dot(x, w2_ref[...], preferred_element_type=jnp.float32)
        + b2_ref[...][None, :], 0.0)
    x = (jnp.dot(x, w3_ref[...], preferred_element_type=jnp.float32)
         + b3_ref[...][None, :])
    o_ref[...] = x


def _tc_pre(h_pad, emb, W1, b1, W2, b2, W3, b3):
    grid = (NP // BLK,)
    full_w = pl.BlockSpec((D, D), lambda i: (0, 0))
    full_b = pl.BlockSpec((D,), lambda i: (0,))
    return pl.pallas_call(
        _tc_pre_body,
        grid=grid,
        in_specs=[
            pl.BlockSpec((1, 1, BLK), lambda i: (i, 0, 0)),
            full_w, full_w, full_b, full_w, full_b, full_w, full_b,
        ],
        out_specs=pl.BlockSpec((BLK, D), lambda i: (i, 0)),
        out_shape=jax.ShapeDtypeStruct((NP, D), jnp.float32),
    )(h_pad, emb, W1, b1, W2, b2, W3, b3)


def _tc_readout_body(x_ref, r0w_ref, r0b_ref, r1w_ref, r1b_ref, r2w_ref,
                     r2b_ref, o_ref):
    y = jnp.maximum(
        jnp.dot(x_ref[...], r0w_ref[...], preferred_element_type=jnp.float32)
        + r0b_ref[...][None, :], 0.0)
    y = jnp.maximum(
        jnp.dot(y, r1w_ref[...], preferred_element_type=jnp.float32)
        + r1b_ref[...][None, :], 0.0)
    y = (jnp.dot(y, r2w_ref[...], preferred_element_type=jnp.float32)
         + r2b_ref[...][None, :])
    o_ref[...] = y


def _tc_readout(x, R0w, R0b, R1w, R1b, R2w, R2b):
    grid = (NP // BLK,)
    return pl.pallas_call(
        _tc_readout_body,
        grid=grid,
        in_specs=[
            pl.BlockSpec((BLK, D), lambda i: (i, 0)),
            pl.BlockSpec((D, 64), lambda i: (0, 0)),
            pl.BlockSpec((64,), lambda i: (0,)),
            pl.BlockSpec((64, 32), lambda i: (0, 0)),
            pl.BlockSpec((32,), lambda i: (0,)),
            pl.BlockSpec((32, 16), lambda i: (0, 0)),
            pl.BlockSpec((16,), lambda i: (0,)),
        ],
        out_specs=pl.BlockSpec((BLK, 16), lambda i: (i, 0)),
        out_shape=jax.ShapeDtypeStruct((NP, 16), jnp.float32),
    )(x, R0w, R0b, R1w, R1b, R2w, R2b)


def _sc_body(src_hbm, dst_hbm, lp_hbm, hw_hbm, hb_hbm,
             x_hbm, steps_hbm, rem_hbm, pa_hbm,
             sbuf, dbuf, sbuf2, dbuf2, sbufB, dbufB1, rows, rows2,
             degb5, dinvb, dscaleb, sumhb, contb, stepsb, rembuf,
             dotb, pbuf, ocontb, hwbuf, hbbuf,
             spdeg, agg_sh, gsem, gsem2):
    cid = lax.axis_index("c")
    sid = lax.axis_index("s")

    @pl.when(cid == 0)
    def _work():
        nbase = sid * ROWS_PER_TILE
        ebase = sid * EDGES_PER_TILE
        ngroups = ROWS_PER_TILE // 16
        zero16 = jnp.zeros((16,), jnp.float32)
        one16 = jnp.full((16,), 1.0, jnp.float32)

        def _zero_rows():
            @pl.loop(0, EW)
            def _zr(i):
                for k in range(8):
                    rows[i, pl.ds(k * 16, 16)] = zero16

        _zero_rows()

        @pl.loop(0, ngroups)
        def _st(i):
            sl = pl.ds(i * 16, 16)
            sumhb[sl] = zero16
            contb[sl] = one16
            stepsb[sl] = one16

        pltpu.sync_copy(hw_hbm, hwbuf)
        pltpu.sync_copy(hb_hbm, hbbuf)

        pltpu.sync_copy(rows.at[pl.ds(0, 5)], spdeg.at[pl.ds(sid * 5, 5)])
        plsc.subcore_barrier()

        @pl.loop(0, EDGES_PER_TILE // 768)
        def _degg(g):
            geb = ebase + g * 768
            pltpu.sync_copy(dst_hbm.at[pl.ds(geb, 768)], dbufB1)
            for w in range(6):
                for q in range(8):
                    sl = pl.ds(q * 16, 16)
                    v = dbufB1[pl.ds(w * 128 + q * 16, 16)]
                    sbuf[sl] = lax.shift_right_logical(v, 7)
                    dbuf[sl] = lax.bitwise_and(v, 127)

                @pl.loop(0, 128)
                def _oh(j):
                    jj = jnp.zeros((16,), jnp.int32) + j
                    lw = plsc.load_gather(dbuf, [jj])
                    plsc.store_scatter(rows, [jj, lw], one16)

                pltpu.sync_copy(rows, spdeg.at[sbuf], add=True)

                @pl.loop(0, 128)
                def _cl(j):
                    jj = jnp.zeros((16,), jnp.int32) + j
                    lw = plsc.load_gather(dbuf, [jj])
                    plsc.store_scatter(rows, [jj, lw], zero16)

        plsc.subcore_barrier()

        pltpu.sync_copy(spdeg.at[pl.ds(sid * 5, 5)], degb5.at[pl.ds(0, 5)])

        @pl.loop(0, 5)
        def _dinv(q):
            for k in range(8):
                deg = jnp.maximum(degb5[q, pl.ds(k * 16, 16)], 1.0)
                ii = lax.bitcast_convert_type(deg, jnp.int32)
                ii = jnp.int32(0x5F3759DF) - lax.shift_right_arithmetic(ii, 1)
                y = lax.bitcast_convert_type(ii, jnp.float32)
                half = deg * 0.5
                for _ in range(4):
                    y = y * (1.5 - half * y * y)
                sl = pl.ds(q * 128 + k * 16, 16)
                dinvb[sl] = y
                dscaleb[sl] = deg * y

        for c in range(CHUNKS):
            rb = nbase + c * NC
            sl_rows = pl.ds(rb, NC)
            pltpu.sync_copy(lp_hbm.at[sl_rows], rows2.at[pl.ds(0, NC)])

            @pl.loop(0, NC)
            def _scale(i, _c=c):
                bidx = jnp.zeros((16,), jnp.int32) + (_c * NC + i)
                dv = plsc.load_gather(dinvb, [bidx])
                for k in range(8):
                    sl = pl.ds(k * 16, 16)
                    rows2[i, sl] = rows2[i, sl] * dv

            pltpu.sync_copy(rows2.at[pl.ds(0, NC)], pa_hbm.at[sl_rows])
            pltpu.sync_copy(rows.at[pl.ds(0, NC)], x_hbm.at[sl_rows])

        plsc.subcore_barrier()

        hw = [hwbuf[pl.ds(k * 16, 16)] for k in range(8)]
        hb16 = hbbuf[...]

        @pl.loop(0, N_ITER)
        def _iter(t):
            _zero_rows()
            for c in range(5):
                pltpu.sync_copy(rows, agg_sh.at[pl.ds(nbase + c * EW, EW)])
            plsc.subcore_barrier()

            sets = [(sbuf, dbuf, rows, gsem), (sbuf2, dbuf2, rows2, gsem2)]

            @pl.loop(0, WINDOWS // 6)
            def _prop(g):
                geb = ebase + g * 768
                pltpu.sync_copy(src_hbm.at[pl.ds(geb, 768)], sbufB)
                pltpu.sync_copy(dst_hbm.at[pl.ds(geb, 768)], dbufB1)

                def _repack(k):
                    sb, db = sets[k % 2][0], sets[k % 2][1]
                    for q in range(8):
                        sl = pl.ds(q * 16, 16)
                        sb[sl] = sbufB[pl.ds(k * 128 + q * 16, 16)]
                        db[sl] = dbufB1[pl.ds(k * 128 + q * 16, 16)]

                _repack(0)
                pltpu.make_async_copy(pa_hbm.at[sbuf], rows, gsem).start()
                for k in range(6):
                    sb, db, rw, sem = sets[k % 2]
                    if k + 1 < 6:
                        _repack(k + 1)
                    pltpu.make_async_copy(pa_hbm.at[sb], rw, sem).wait()
                    if k + 1 < 6:
                        sb2, db2, rw2, sem2 = sets[(k + 1) % 2]
                        pltpu.make_async_copy(
                            pa_hbm.at[sb2], rw2, sem2).start()
                    pltpu.sync_copy(rw, agg_sh.at[db], add=True)

            plsc.subcore_barrier()

            for c in range(CHUNKS):
                rb = nbase + c * NC
                sl_rows = pl.ds(rb, NC)
                pltpu.sync_copy(agg_sh.at[sl_rows], rows2.at[pl.ds(0, NC)])
                pltpu.sync_copy(pa_hbm.at[sl_rows], rows2.at[pl.ds(NC, NC)])
                pltpu.sync_copy(x_hbm.at[sl_rows], rows.at[pl.ds(0, NC)])

                @pl.loop(0, NC)
                def _p1(i, _c=c):
                    bidx = jnp.zeros((16,), jnp.int32) + (_c * NC + i)
                    dv = plsc.load_gather(dinvb, [bidx])
                    acc = zero16
                    for k in range(8):
                        sl = pl.ds(k * 16, 16)
                        a = rows2[i, sl] * dv
                        rows2[i, sl] = a
                        acc = acc + a * hw[k]
                    plsc.store_scatter(
                        dotb, [jnp.zeros((16,), jnp.int32) + i],
                        one16 * jnp.sum(acc))

                @pl.loop(0, NC // 16)
                def _hc(gidx, _c=c):
                    lsl = pl.ds(gidx * 16, 16)
                    gsl = pl.ds(_c * NC + gidx * 16, 16)
                    d16 = dotb[lsl]
                    z = d16 + hb16
                    hh = 1.0 / (1.0 + jnp.exp(-z))
                    sh = sumhb[gsl]
                    ct = contb[gsl]
                    st = stepsb[gsl]
                    pm = jnp.where(sh + hh < 0.99, 1.0, 0.0) * ct
                    st = st + pm
                    sh = sh + pm * hh
                    fi = jnp.where(st < float(N_ITER), 1.0, 0.0)
                    cond = pm * fi
                    p = jnp.where(cond > 0.0, sh, 1.0 - sh)
                    stepsb[gsl] = st
                    sumhb[gsl] = sh
                    contb[gsl] = ct * pm
                    pbuf[lsl] = p
                    ocontb[lsl] = ct

                @pl.loop(0, NC)
                def _p2(i, _c=c):
                    gidx = jnp.zeros((16,), jnp.int32) + (_c * NC + i)
                    lidx = jnp.zeros((16,), jnp.int32) + i
                    dv = plsc.load_gather(dinvb, [gidx])
                    dsc = plsc.load_gather(dscaleb, [gidx])
                    p_s = plsc.load_gather(pbuf, [lidx])
                    sh_s = plsc.load_gather(sumhb, [gidx])
                    ct_s = plsc.load_gather(ocontb, [lidx])
                    for k in range(8):
                        sl = pl.ds(k * 16, 16)
                        new = rows2[i, sl]
                        old = rows2[NC + i, sl] * dsc
                        rows[i, sl] = rows[i, sl] + (new * p_s - old * sh_s) * ct_s
                        rows2[NC + i, sl] = new * dv

                pltpu.sync_copy(rows2.at[pl.ds(NC, NC)], pa_hbm.at[sl_rows])
                pltpu.sync_copy(rows.at[pl.ds(0, NC)], x_hbm.at[sl_rows])

            plsc.subcore_barrier()

        @pl.loop(0, ngroups)
        def _rem(i):
            sl = pl.ds(i * 16, 16)
            rembuf[sl] = 1.0 - sumhb[sl]

        pltpu.sync_copy(stepsb, steps_hbm.at[pl.ds(nbase, ROWS_PER_TILE)])
        pltpu.sync_copy(rembuf, rem_hbm.at[pl.ds(nbase, ROWS_PER_TILE)])


def _sc_prop(src_all, dst_all, local_preds, hw, hb16):
    mesh = plsc.VectorSubcoreMesh(core_axis_name="c", subcore_axis_name="s")
    f32 = jnp.float32
    cp = pltpu.CompilerParams()
    if "needs_layout_passes" in pltpu.CompilerParams.__dataclass_fields__:
        cp = dataclasses.replace(cp, needs_layout_passes=False)
    kern = pl.kernel(
        _sc_body,
        compiler_params=cp,
        out_type=[
            jax.ShapeDtypeStruct((NP, D), f32),
            jax.ShapeDtypeStruct((NP,), f32),
            jax.ShapeDtypeStruct((NP,), f32),
            jax.ShapeDtypeStruct((NP, D), f32),
        ],
        mesh=mesh,
        scratch_types=[
            pltpu.VMEM((EW,), jnp.int32),
            pltpu.VMEM((EW,), jnp.int32),
            pltpu.VMEM((EW,), jnp.int32),
            pltpu.VMEM((EW,), jnp.int32),
            pltpu.VMEM((768,), jnp.int32),
            pltpu.VMEM((768,), jnp.int32),
            pltpu.VMEM((EW, D), f32),
            pltpu.VMEM((EW, D), f32),
            pltpu.VMEM((8, D), f32),
            pltpu.VMEM((ROWS_PER_TILE,), f32),
            pltpu.VMEM((ROWS_PER_TILE,), f32),
            pltpu.VMEM((ROWS_PER_TILE,), f32),
            pltpu.VMEM((ROWS_PER_TILE,), f32),
            pltpu.VMEM((ROWS_PER_TILE,), f32),
            pltpu.VMEM((ROWS_PER_TILE,), f32),
            pltpu.VMEM((NC,), f32),
            pltpu.VMEM((NC,), f32),
            pltpu.VMEM((NC,), f32),
            pltpu.VMEM((D,), f32),
            pltpu.VMEM((16,), f32),
            pltpu.VMEM_SHARED((NP // 128, D), f32),
            pltpu.VMEM_SHARED((NP, D), f32),
            pltpu.SemaphoreType.DMA,
            pltpu.SemaphoreType.DMA,
        ],
    )
    return kern(src_all, dst_all, local_preds, hw, hb16)


def kernel(g, h, e, snorm_n, snorm_e, emb, W1, b1, W2, b2, W3, b3,
           halt_w, halt_b, R0w, R0b, R1w, R1b, R2w, R2b):
    loop_idx = jnp.arange(N, dtype=jnp.int32)
    pad_idx = jnp.full((PAD_EDGES,), NP - 1, jnp.int32)
    src_all = jnp.concatenate([g[0], loop_idx, pad_idx])
    dst_all = jnp.concatenate([g[1], loop_idx, pad_idx])
    h_pad = jnp.concatenate([h, jnp.zeros((NP - N,), jnp.int32)])
    h_pad = h_pad.reshape(NP // BLK, 1, BLK)
    hw = halt_w.reshape(D)
    hb16 = jnp.full((16,), halt_b[0], jnp.float32)

    local_preds = _tc_pre(h_pad, emb, W1, b1, W2, b2, W3, b3)
    x, steps, rem, _ = _sc_prop(src_all, dst_all, local_preds, hw, hb16)
    y = _tc_readout(x, R0w, R0b, R1w, R1b, R2w, R2b)
    return y[:N], steps[:N], rem[:N]
